# baseline (speedup 1.0000x reference)
import functools

import jax
import jax.numpy as jnp
from jax import lax
from jax.experimental import pallas as pl
from jax.experimental.pallas import tpu as pltpu
from jax.experimental.pallas import tpu_sc as plsc

PMT = 16
D = 64


def _sc_gather(table, idx):
    info = plsc.get_sparse_core_info()
    nw = info.num_cores * info.num_subcores
    b = idx.shape[0]
    b_per_w = b // nw
    mesh = plsc.VectorSubcoreMesh(core_axis_name="c", subcore_axis_name="s")

    @functools.partial(
        pl.kernel,
        mesh=mesh,
        out_type=jax.ShapeDtypeStruct((b, D), jnp.float32),
        scratch_types=[
            pltpu.VMEM((b_per_w,), jnp.int32),
            pltpu.VMEM((b_per_w, D), jnp.float32),
            pltpu.SemaphoreType.DMA,
        ],
        compiler_params=pltpu.CompilerParams(use_tc_tiling_on_sc=False),
    )
    def k(table_hbm, idx_hbm, out_hbm, idx_v, rows_v, sem):
        wid = lax.axis_index("s") * info.num_cores + lax.axis_index("c")
        base = wid * b_per_w
        pltpu.sync_copy(idx_hbm.at[pl.ds(base, b_per_w)], idx_v)
        pltpu.async_copy(table_hbm.at[idx_v], rows_v, sem).wait()
        pltpu.sync_copy(rows_v, out_hbm.at[pl.ds(base, b_per_w)])

    return k(table, idx)


def _tc_assemble(src_embs, src_mask, cell_vec, prompt_base):
    bs, seq, d = src_embs.shape
    tot = PMT + seq
    bb = 128
    grid = (bs // bb,)

    def body(src_ref, mask_ref, cv_ref, pb_ref, x_ref, m_ref):
        x_ref[:, :PMT, :] = pb_ref[...][None, :, :] + cv_ref[...][:, None, :]
        x_ref[:, PMT:, :] = src_ref[...]
        m_ref[:, :PMT] = jnp.ones((bb, PMT), dtype=m_ref.dtype)
        m_ref[:, PMT:] = mask_ref[...]

    return pl.pallas_call(
        body,
        grid=grid,
        in_specs=[
            pl.BlockSpec((bb, seq, d), lambda i: (i, 0, 0)),
            pl.BlockSpec((bb, seq), lambda i: (i, 0)),
            pl.BlockSpec((bb, d), lambda i: (i, 0)),
            pl.BlockSpec((PMT, d), lambda i: (0, 0)),
        ],
        out_specs=[
            pl.BlockSpec((bb, tot, d), lambda i: (i, 0, 0)),
            pl.BlockSpec((bb, tot), lambda i: (i, 0)),
        ],
        out_shape=[
            jax.ShapeDtypeStruct((bs, tot, d), src_embs.dtype),
            jax.ShapeDtypeStruct((bs, tot), src_mask.dtype),
        ],
    )(src_embs, src_mask, cell_vec, prompt_base)


def kernel(src_embs, src_mask, cell_idx, prompt_base, cell_embed_weight):
    cell_vec = _sc_gather(cell_embed_weight, cell_idx.astype(jnp.int32))
    x, new_mask = _tc_assemble(src_embs, src_mask, cell_vec, prompt_base)
    return (x, new_mask)


# speedup vs baseline: 3.0834x; 3.0834x over previous
import jax
import jax.numpy as jnp
from jax.experimental import pallas as pl
from jax.experimental.pallas import tpu as pltpu

PMT = 16
D = 64
KPG = 8


def _tc_gather(table_t, cell_idx):
    d, v = table_t.shape
    b = cell_idx.shape[0]
    grid = (b // KPG,)

    def body(idx_ref, *refs):
        tbl_refs = refs[:KPG]
        out_ref = refs[KPG]
        i = pl.program_id(0)
        lane = jax.lax.broadcasted_iota(jnp.int32, (d, 128), 1)
        for k in range(KPG):
            col = idx_ref[i * KPG + k] % 128
            x = tbl_refs[k][...]
            out_ref[k, :] = jnp.sum(jnp.where(lane == col, x, 0.0), axis=1)

    tbl_spec = lambda k: pl.BlockSpec(
        (d, 128), lambda i, idx_ref, k=k: (0, idx_ref[i * KPG + k] // 128))
    return pl.pallas_call(
        body,
        grid_spec=pltpu.PrefetchScalarGridSpec(
            num_scalar_prefetch=1,
            grid=grid,
            in_specs=[tbl_spec(k) for k in range(KPG)],
            out_specs=pl.BlockSpec((KPG, d), lambda i, idx_ref: (i, 0)),
        ),
        out_shape=jax.ShapeDtypeStruct((b, d), jnp.float32),
    )(cell_idx, *([table_t] * KPG))


def _tc_assemble(src_t, mask_t, cell_vec_t, prompt_base):
    seq, d, b = src_t.shape
    tot = PMT + seq
    tb = 8
    npmt = PMT // tb
    grid = (tot // tb,)

    def body(src_ref, mask_ref, cvt_ref, pb_ref, x_ref, m_ref):
        i = pl.program_id(0)

        @pl.when(i < npmt)
        def _():
            x_ref[...] = pb_ref[...][:, :, None] + cvt_ref[...][None, :, :]
            m_ref[...] = jnp.ones((tb, b), m_ref.dtype)

        @pl.when(i >= npmt)
        def _():
            x_ref[...] = src_ref[...]
            m_ref[...] = mask_ref[...]

    return pl.pallas_call(
        body,
        grid=grid,
        in_specs=[
            pl.BlockSpec((tb, d, b),
                         lambda i: (jnp.maximum(i - npmt, 0), 0, 0)),
            pl.BlockSpec((tb, b), lambda i: (jnp.maximum(i - npmt, 0), 0)),
            pl.BlockSpec((d, b), lambda i: (0, 0)),
            pl.BlockSpec((tb, d), lambda i: (jnp.minimum(i, npmt - 1), 0)),
        ],
        out_specs=[
            pl.BlockSpec((tb, d, b), lambda i: (i, 0, 0)),
            pl.BlockSpec((tb, b), lambda i: (i, 0)),
        ],
        out_shape=[
            jax.ShapeDtypeStruct((tot, d, b), src_t.dtype),
            jax.ShapeDtypeStruct((tot, b), mask_t.dtype),
        ],
    )(src_t, mask_t, cell_vec_t, prompt_base)


def kernel(src_embs, src_mask, cell_idx, prompt_base, cell_embed_weight):
    table_t = cell_embed_weight.T
    src_t = jnp.transpose(src_embs, (1, 2, 0))
    mask_t = src_mask.T
    cell_vec = _tc_gather(table_t, cell_idx.astype(jnp.int32))
    x_t, new_mask_t = _tc_assemble(src_t, mask_t, cell_vec.T, prompt_base)
    x = jnp.transpose(x_t, (2, 0, 1))
    new_mask = new_mask_t.T
    return (x, new_mask)


# speedup vs baseline: 6.4998x; 2.1080x over previous
import functools

import jax
import jax.numpy as jnp
from jax import lax
from jax.experimental import pallas as pl
from jax.experimental.pallas import tpu as pltpu
from jax.experimental.pallas import tpu_sc as plsc

PMT = 16
D = 64
KPG = 8


def _sc_gather(table_t, cell_idx):
    d, v = table_t.shape
    b = cell_idx.shape[0]
    info = plsc.get_sparse_core_info()
    nw = info.num_cores * info.num_subcores
    bpw = b // nw
    chunk = 8
    mesh = plsc.VectorSubcoreMesh(core_axis_name="c", subcore_axis_name="s")

    @functools.partial(
        pl.kernel,
        mesh=mesh,
        out_type=jax.ShapeDtypeStruct((d, b), jnp.float32),
        scratch_types=[
            pltpu.VMEM((bpw,), jnp.int32),
            pltpu.VMEM((d, chunk * 128), jnp.float32),
            pltpu.VMEM((d, bpw), jnp.float32),
            pltpu.SemaphoreType.DMA,
        ],
        compiler_params=pltpu.CompilerParams(use_tc_tiling_on_sc=True,
                                             needs_layout_passes=False),
    )
    def k(table_hbm, idx_hbm, out_hbm, idx_v, stage_v, tile_v, sem):
        wid = lax.axis_index("s") * info.num_cores + lax.axis_index("c")
        base = wid * bpw
        pltpu.sync_copy(idx_hbm.at[pl.ds(base, bpw)], idx_v)

        def body(j0, carry):
            vec = idx_v[pl.ds(j0 * 16, 16)]
            for half in range(2):
                for s in range(chunk):
                    t = half * chunk + s
                    col0 = pl.multiple_of((vec[t] // 128) * 128, 128)
                    pltpu.async_copy(
                        table_hbm.at[:, pl.ds(col0, 128)],
                        stage_v.at[:, pl.ds(s * 128, 128)],
                        sem,
                    )
                for s in range(chunk):
                    pltpu.make_async_copy(
                        table_hbm.at[:, pl.ds(0, 128)],
                        stage_v.at[:, pl.ds(s * 128, 128)],
                        sem,
                    ).wait()
                for s in range(chunk):
                    t = half * chunk + s
                    lane = vec[t] % 128
                    j = j0 * 16 + t
                    for g in range(d // 16):
                        rows = jnp.arange(16, dtype=jnp.int32) + g * 16
                        vals = plsc.load_gather(
                            stage_v, [rows, jnp.full((16,), s * 128 + lane,
                                                     jnp.int32)])
                        plsc.store_scatter(
                            tile_v, [rows, jnp.full((16,), j, jnp.int32)],
                            vals)
            return carry

        lax.fori_loop(0, bpw // 16, body, 0)
        pltpu.sync_copy(tile_v, out_hbm.at[:, pl.ds(base, bpw)])

    return k(table_t, cell_idx)


def _tc_gather(table_t, cell_idx):
    d, v = table_t.shape
    b = cell_idx.shape[0]
    grid = (b // KPG,)

    def body(idx_ref, *refs):
        tbl_refs = refs[:KPG]
        out_ref = refs[KPG]
        i = pl.program_id(0)
        lane = jax.lax.broadcasted_iota(jnp.int32, (d, 128), 1)
        for k in range(KPG):
            col = idx_ref[i * KPG + k] % 128
            x = tbl_refs[k][...]
            out_ref[k, :] = jnp.sum(jnp.where(lane == col, x, 0.0), axis=1)

    tbl_spec = lambda k: pl.BlockSpec(
        (d, 128), lambda i, idx_ref, k=k: (0, idx_ref[i * KPG + k] // 128))
    return pl.pallas_call(
        body,
        grid_spec=pltpu.PrefetchScalarGridSpec(
            num_scalar_prefetch=1,
            grid=grid,
            in_specs=[tbl_spec(k) for k in range(KPG)],
            out_specs=pl.BlockSpec((KPG, d), lambda i, idx_ref: (i, 0)),
        ),
        out_shape=jax.ShapeDtypeStruct((b, d), jnp.float32),
    )(cell_idx, *([table_t] * KPG))


def _tc_assemble(src_t, mask_t, cell_vec_t, prompt_base):
    seq, d, b = src_t.shape
    tot = PMT + seq
    tb = 8
    npmt = PMT // tb
    grid = (tot // tb,)

    def body(src_ref, mask_ref, cvt_ref, pb_ref, x_ref, m_ref):
        i = pl.program_id(0)

        @pl.when(i < npmt)
        def _():
            x_ref[...] = pb_ref[...][:, :, None] + cvt_ref[...][None, :, :]
            m_ref[...] = jnp.ones((tb, b), m_ref.dtype)

        @pl.when(i >= npmt)
        def _():
            x_ref[...] = src_ref[...]
            m_ref[...] = mask_ref[...]

    return pl.pallas_call(
        body,
        grid=grid,
        in_specs=[
            pl.BlockSpec((tb, d, b),
                         lambda i: (jnp.maximum(i - npmt, 0), 0, 0)),
            pl.BlockSpec((tb, b), lambda i: (jnp.maximum(i - npmt, 0), 0)),
            pl.BlockSpec((d, b), lambda i: (0, 0)),
            pl.BlockSpec((tb, d), lambda i: (jnp.minimum(i, npmt - 1), 0)),
        ],
        out_specs=[
            pl.BlockSpec((tb, d, b), lambda i: (i, 0, 0)),
            pl.BlockSpec((tb, b), lambda i: (i, 0)),
        ],
        out_shape=[
            jax.ShapeDtypeStruct((tot, d, b), src_t.dtype),
            jax.ShapeDtypeStruct((tot, b), mask_t.dtype),
        ],
    )(src_t, mask_t, cell_vec_t, prompt_base)


def kernel(src_embs, src_mask, cell_idx, prompt_base, cell_embed_weight):
    table_t = cell_embed_weight.T
    src_t = jnp.transpose(src_embs, (1, 2, 0))
    mask_t = src_mask.T
    cell_vec_t = _sc_gather(table_t, cell_idx.astype(jnp.int32))
    x_t, new_mask_t = _tc_assemble(src_t, mask_t, cell_vec_t, prompt_base)
    x = jnp.transpose(x_t, (2, 0, 1))
    new_mask = new_mask_t.T
    return (x, new_mask)


# speedup vs baseline: 7.0640x; 1.0868x over previous
import functools

import jax
import jax.numpy as jnp
from jax import lax
from jax.experimental import pallas as pl
from jax.experimental.pallas import tpu as pltpu
from jax.experimental.pallas import tpu_sc as plsc

PMT = 16
D = 64
KPG = 8


def _sc_gather(table_t, cell_idx):
    d, v = table_t.shape
    b = cell_idx.shape[0]
    info = plsc.get_sparse_core_info()
    nw = info.num_cores * info.num_subcores
    bpw = b // nw
    chunk = 8
    mesh = plsc.VectorSubcoreMesh(core_axis_name="c", subcore_axis_name="s")

    @functools.partial(
        pl.kernel,
        mesh=mesh,
        out_type=jax.ShapeDtypeStruct((d, b), jnp.float32),
        scratch_types=[
            pltpu.VMEM((bpw,), jnp.int32),
            pltpu.VMEM((d, chunk * 128), jnp.float32),
            pltpu.VMEM((d, bpw), jnp.float32),
            pltpu.SemaphoreType.DMA,
        ],
        compiler_params=pltpu.CompilerParams(use_tc_tiling_on_sc=True,
                                             needs_layout_passes=False),
    )
    def k(table_hbm, idx_hbm, out_hbm, idx_v, stage_v, tile_v, sem):
        wid = lax.axis_index("s") * info.num_cores + lax.axis_index("c")
        base = wid * bpw
        pltpu.sync_copy(idx_hbm.at[pl.ds(base, bpw)], idx_v)

        def body(j0, carry):
            vec = idx_v[pl.ds(j0 * 16, 16)]
            for half in range(2):
                for s in range(chunk):
                    t = half * chunk + s
                    col0 = pl.multiple_of((vec[t] // 128) * 128, 128)
                    pltpu.async_copy(
                        table_hbm.at[:, pl.ds(col0, 128)],
                        stage_v.at[:, pl.ds(s * 128, 128)],
                        sem,
                    )
                for s in range(chunk):
                    pltpu.make_async_copy(
                        table_hbm.at[:, pl.ds(0, 128)],
                        stage_v.at[:, pl.ds(s * 128, 128)],
                        sem,
                    ).wait()
                for s in range(chunk):
                    t = half * chunk + s
                    lane = vec[t] % 128
                    j = j0 * 16 + t
                    for g in range(d // 16):
                        rows = jnp.arange(16, dtype=jnp.int32) + g * 16
                        vals = plsc.load_gather(
                            stage_v, [rows, jnp.full((16,), s * 128 + lane,
                                                     jnp.int32)])
                        plsc.store_scatter(
                            tile_v, [rows, jnp.full((16,), j, jnp.int32)],
                            vals)
            return carry

        lax.fori_loop(0, bpw // 16, body, 0)
        pltpu.sync_copy(tile_v, out_hbm.at[:, pl.ds(base, bpw)])

    return k(table_t, cell_idx)


def _tc_gather(table_t, cell_idx):
    d, v = table_t.shape
    b = cell_idx.shape[0]
    grid = (b // KPG,)

    def body(idx_ref, *refs):
        tbl_refs = refs[:KPG]
        out_ref = refs[KPG]
        i = pl.program_id(0)
        lane = jax.lax.broadcasted_iota(jnp.int32, (d, 128), 1)
        for k in range(KPG):
            col = idx_ref[i * KPG + k] % 128
            x = tbl_refs[k][...]
            out_ref[k, :] = jnp.sum(jnp.where(lane == col, x, 0.0), axis=1)

    tbl_spec = lambda k: pl.BlockSpec(
        (d, 128), lambda i, idx_ref, k=k: (0, idx_ref[i * KPG + k] // 128))
    return pl.pallas_call(
        body,
        grid_spec=pltpu.PrefetchScalarGridSpec(
            num_scalar_prefetch=1,
            grid=grid,
            in_specs=[tbl_spec(k) for k in range(KPG)],
            out_specs=pl.BlockSpec((KPG, d), lambda i, idx_ref: (i, 0)),
        ),
        out_shape=jax.ShapeDtypeStruct((b, d), jnp.float32),
    )(cell_idx, *([table_t] * KPG))


def _tc_src_copy(src_t, mask_t):
    seq, d, b = src_t.shape
    tot = PMT + seq
    tb = 8
    grid = (seq // tb,)
    off = PMT // tb

    def body(src_ref, mask_ref, x_ref, m_ref):
        x_ref[...] = src_ref[...]
        m_ref[...] = mask_ref[...]

    return pl.pallas_call(
        body,
        grid=grid,
        in_specs=[
            pl.BlockSpec((tb, d, b), lambda i: (i, 0, 0)),
            pl.BlockSpec((tb, b), lambda i: (i, 0)),
        ],
        out_specs=[
            pl.BlockSpec((tb, d, b), lambda i: (i + off, 0, 0)),
            pl.BlockSpec((tb, b), lambda i: (i + off, 0)),
        ],
        out_shape=[
            jax.ShapeDtypeStruct((tot, d, b), src_t.dtype),
            jax.ShapeDtypeStruct((tot, b), mask_t.dtype),
        ],
    )(src_t, mask_t)


def _tc_prompt_fill(x_part, m_part, cell_vec_t, prompt_base):
    tot, d, b = x_part.shape
    tb = 8
    grid = (PMT // tb,)

    def body(x_in, m_in, cvt_ref, pb_ref, x_ref, m_ref):
        x_ref[...] = pb_ref[...][:, :, None] + cvt_ref[...][None, :, :]
        m_ref[...] = jnp.ones((tb, b), m_ref.dtype)

    return pl.pallas_call(
        body,
        grid=grid,
        in_specs=[
            pl.BlockSpec(memory_space=pl.ANY),
            pl.BlockSpec(memory_space=pl.ANY),
            pl.BlockSpec((d, b), lambda i: (0, 0)),
            pl.BlockSpec((tb, d), lambda i: (i, 0)),
        ],
        out_specs=[
            pl.BlockSpec((tb, d, b), lambda i: (i, 0, 0)),
            pl.BlockSpec((tb, b), lambda i: (i, 0)),
        ],
        out_shape=[
            jax.ShapeDtypeStruct((tot, d, b), x_part.dtype),
            jax.ShapeDtypeStruct((tot, b), m_part.dtype),
        ],
        input_output_aliases={0: 0, 1: 1},
    )(x_part, m_part, cell_vec_t, prompt_base)


def _tc_assemble(src_t, mask_t, cell_vec_t, prompt_base):
    seq, d, b = src_t.shape
    tot = PMT + seq
    tb = 8
    npmt = PMT // tb
    grid = (tot // tb,)

    def body(src_ref, mask_ref, cvt_ref, pb_ref, x_ref, m_ref):
        i = pl.program_id(0)

        @pl.when(i < npmt)
        def _():
            x_ref[...] = pb_ref[...][:, :, None] + cvt_ref[...][None, :, :]
            m_ref[...] = jnp.ones((tb, b), m_ref.dtype)

        @pl.when(i >= npmt)
        def _():
            x_ref[...] = src_ref[...]
            m_ref[...] = mask_ref[...]

    return pl.pallas_call(
        body,
        grid=grid,
        in_specs=[
            pl.BlockSpec((tb, d, b),
                         lambda i: (jnp.maximum(i - npmt, 0), 0, 0)),
            pl.BlockSpec((tb, b), lambda i: (jnp.maximum(i - npmt, 0), 0)),
            pl.BlockSpec((d, b), lambda i: (0, 0)),
            pl.BlockSpec((tb, d), lambda i: (jnp.minimum(i, npmt - 1), 0)),
        ],
        out_specs=[
            pl.BlockSpec((tb, d, b), lambda i: (i, 0, 0)),
            pl.BlockSpec((tb, b), lambda i: (i, 0)),
        ],
        out_shape=[
            jax.ShapeDtypeStruct((tot, d, b), src_t.dtype),
            jax.ShapeDtypeStruct((tot, b), mask_t.dtype),
        ],
    )(src_t, mask_t, cell_vec_t, prompt_base)


def kernel(src_embs, src_mask, cell_idx, prompt_base, cell_embed_weight):
    table_t = cell_embed_weight.T
    src_t = jnp.transpose(src_embs, (1, 2, 0))
    mask_t = src_mask.T
    cell_vec_t = _sc_gather(table_t, cell_idx.astype(jnp.int32))
    x_p, m_p = _tc_src_copy(src_t, mask_t)
    x_t, new_mask_t = _tc_prompt_fill(x_p, m_p, cell_vec_t, prompt_base)
    x = jnp.transpose(x_t, (2, 0, 1))
    new_mask = new_mask_t.T
    return (x, new_mask)


# speedup vs baseline: 7.4960x; 1.0612x over previous
import functools

import jax
import jax.numpy as jnp
from jax import lax
from jax.experimental import pallas as pl
from jax.experimental.pallas import tpu as pltpu
from jax.experimental.pallas import tpu_sc as plsc

PMT = 16
D = 64
KPG = 8


def _sc_gather(table_t, cell_idx):
    d, v = table_t.shape
    b = cell_idx.shape[0]
    info = plsc.get_sparse_core_info()
    nw = info.num_cores * info.num_subcores
    bpw = b // nw
    chunk = 8
    mesh = plsc.VectorSubcoreMesh(core_axis_name="c", subcore_axis_name="s")

    @functools.partial(
        pl.kernel,
        mesh=mesh,
        out_type=jax.ShapeDtypeStruct((d, b), jnp.float32),
        scratch_types=[
            pltpu.VMEM((bpw,), jnp.int32),
            pltpu.VMEM((d, chunk * 128), jnp.float32),
            pltpu.VMEM((d, bpw), jnp.float32),
            pltpu.SemaphoreType.DMA,
        ],
        compiler_params=pltpu.CompilerParams(use_tc_tiling_on_sc=True,
                                             needs_layout_passes=False),
    )
    def k(table_hbm, idx_hbm, out_hbm, idx_v, stage_v, tile_v, sem):
        wid = lax.axis_index("s") * info.num_cores + lax.axis_index("c")
        base = wid * bpw
        pltpu.sync_copy(idx_hbm.at[pl.ds(base, bpw)], idx_v)

        def body(j0, carry):
            vec = idx_v[pl.ds(j0 * 16, 16)]
            for half in range(2):
                for s in range(chunk):
                    t = half * chunk + s
                    col0 = pl.multiple_of((vec[t] // 128) * 128, 128)
                    pltpu.async_copy(
                        table_hbm.at[:, pl.ds(col0, 128)],
                        stage_v.at[:, pl.ds(s * 128, 128)],
                        sem,
                    )
                for s in range(chunk):
                    pltpu.make_async_copy(
                        table_hbm.at[:, pl.ds(0, 128)],
                        stage_v.at[:, pl.ds(s * 128, 128)],
                        sem,
                    ).wait()
                for s in range(chunk):
                    t = half * chunk + s
                    lane = vec[t] % 128
                    j = j0 * 16 + t
                    for g in range(d // 16):
                        rows = jnp.arange(16, dtype=jnp.int32) + g * 16
                        vals = plsc.load_gather(
                            stage_v, [rows, jnp.full((16,), s * 128 + lane,
                                                     jnp.int32)])
                        plsc.store_scatter(
                            tile_v, [rows, jnp.full((16,), j, jnp.int32)],
                            vals)
            return carry

        lax.fori_loop(0, bpw // 16, body, 0)
        pltpu.sync_copy(tile_v, out_hbm.at[:, pl.ds(base, bpw)])

    return k(table_t, cell_idx)


def _tc_gather(table_t, cell_idx):
    d, v = table_t.shape
    b = cell_idx.shape[0]
    grid = (b // KPG,)

    def body(idx_ref, *refs):
        tbl_refs = refs[:KPG]
        out_ref = refs[KPG]
        i = pl.program_id(0)
        lane = jax.lax.broadcasted_iota(jnp.int32, (d, 128), 1)
        for k in range(KPG):
            col = idx_ref[i * KPG + k] % 128
            x = tbl_refs[k][...]
            out_ref[k, :] = jnp.sum(jnp.where(lane == col, x, 0.0), axis=1)

    tbl_spec = lambda k: pl.BlockSpec(
        (d, 128), lambda i, idx_ref, k=k: (0, idx_ref[i * KPG + k] // 128))
    return pl.pallas_call(
        body,
        grid_spec=pltpu.PrefetchScalarGridSpec(
            num_scalar_prefetch=1,
            grid=grid,
            in_specs=[tbl_spec(k) for k in range(KPG)],
            out_specs=pl.BlockSpec((KPG, d), lambda i, idx_ref: (i, 0)),
        ),
        out_shape=jax.ShapeDtypeStruct((b, d), jnp.float32),
    )(cell_idx, *([table_t] * KPG))


def _tc_src_copy(src_t):
    seq, d, b = src_t.shape
    tot = PMT + seq
    tb = 8
    grid = (seq // tb,)
    off = PMT // tb

    def body(src_ref, x_ref):
        x_ref[...] = src_ref[...]

    return pl.pallas_call(
        body,
        grid=grid,
        in_specs=[
            pl.BlockSpec((tb, d, b), lambda i: (i, 0, 0)),
        ],
        out_specs=pl.BlockSpec((tb, d, b), lambda i: (i + off, 0, 0)),
        out_shape=jax.ShapeDtypeStruct((tot, d, b), src_t.dtype),
    )(src_t)


def _tc_prompt_fill(x_part, cell_vec_t, prompt_base):
    tot, d, b = x_part.shape
    tb = 8
    grid = (PMT // tb,)

    def body(x_in, cvt_ref, pb_ref, x_ref):
        x_ref[...] = pb_ref[...][:, :, None] + cvt_ref[...][None, :, :]

    return pl.pallas_call(
        body,
        grid=grid,
        in_specs=[
            pl.BlockSpec(memory_space=pl.ANY),
            pl.BlockSpec((d, b), lambda i: (0, 0)),
            pl.BlockSpec((tb, d), lambda i: (i, 0)),
        ],
        out_specs=pl.BlockSpec((tb, d, b), lambda i: (i, 0, 0)),
        out_shape=jax.ShapeDtypeStruct((tot, d, b), x_part.dtype),
        input_output_aliases={0: 0},
    )(x_part, cell_vec_t, prompt_base)


def _tc_assemble(src_t, mask_t, cell_vec_t, prompt_base):
    seq, d, b = src_t.shape
    tot = PMT + seq
    tb = 8
    npmt = PMT // tb
    grid = (tot // tb,)

    def body(src_ref, mask_ref, cvt_ref, pb_ref, x_ref, m_ref):
        i = pl.program_id(0)

        @pl.when(i < npmt)
        def _():
            x_ref[...] = pb_ref[...][:, :, None] + cvt_ref[...][None, :, :]
            m_ref[...] = jnp.ones((tb, b), m_ref.dtype)

        @pl.when(i >= npmt)
        def _():
            x_ref[...] = src_ref[...]
            m_ref[...] = mask_ref[...]

    return pl.pallas_call(
        body,
        grid=grid,
        in_specs=[
            pl.BlockSpec((tb, d, b),
                         lambda i: (jnp.maximum(i - npmt, 0), 0, 0)),
            pl.BlockSpec((tb, b), lambda i: (jnp.maximum(i - npmt, 0), 0)),
            pl.BlockSpec((d, b), lambda i: (0, 0)),
            pl.BlockSpec((tb, d), lambda i: (jnp.minimum(i, npmt - 1), 0)),
        ],
        out_specs=[
            pl.BlockSpec((tb, d, b), lambda i: (i, 0, 0)),
            pl.BlockSpec((tb, b), lambda i: (i, 0)),
        ],
        out_shape=[
            jax.ShapeDtypeStruct((tot, d, b), src_t.dtype),
            jax.ShapeDtypeStruct((tot, b), mask_t.dtype),
        ],
    )(src_t, mask_t, cell_vec_t, prompt_base)


def kernel(src_embs, src_mask, cell_idx, prompt_base, cell_embed_weight):
    table_t = cell_embed_weight.T
    src_t = jnp.transpose(src_embs, (1, 2, 0))
    mask_t = src_mask.T
    cell_vec_t = _sc_gather(table_t, cell_idx.astype(jnp.int32))
    x_p = _tc_src_copy(src_t)
    x_t = _tc_prompt_fill(x_p, cell_vec_t, prompt_base)
    x = jnp.transpose(x_t, (2, 0, 1))
    new_mask_t = jnp.concatenate(
        [jnp.ones((PMT, mask_t.shape[1]), mask_t.dtype), mask_t], axis=0)
    new_mask = new_mask_t.T
    return (x, new_mask)
